# baseline (speedup 1.0000x reference)
import jax
import jax.numpy as jnp
from jax import lax
from jax.experimental import pallas as pl
from jax.experimental.pallas import tpu as pltpu
from jax.experimental.pallas import tpu_sc as plsc

_B_TOTAL = 16384 * 200
_D = 64
_V = 1000
_NC, _NS = 2, 16
_NW = _NC * _NS
_B_PER_W = _B_TOTAL // _NW
_GRP = 256
_G = _B_PER_W // _GRP
_BLK = _GRP // 16


def _emb_body(idx_hbm, table_hbm, out_hbm, table_v, idx_v, rows_v, isem, osem):
    sid = lax.axis_index("s")
    wid = sid * _NC + lax.axis_index("c")
    base = wid * _B_PER_W

    pltpu.sync_copy(table_hbm, table_v)

    iota64 = lax.iota(jnp.int32, 16) * 64

    def fire_idx(g, b):
        pltpu.async_copy(idx_hbm.at[pl.ds(base + g * _GRP, _GRP)], idx_v.at[b], isem)

    def wait_idx(b):
        pltpu.make_async_copy(idx_hbm.at[pl.ds(0, _GRP)], idx_v.at[b], isem).wait()

    def fire_out(g, b):
        pltpu.async_copy(
            rows_v.at[b], out_hbm.at[pl.ds((base + g * _GRP) * _D, _GRP * _D)], osem
        )

    def wait_out(b):
        pltpu.make_async_copy(
            rows_v.at[b], out_hbm.at[pl.ds(0, _GRP * _D)], osem
        ).wait()

    def compute_group(b):
        def blk(i, carry):
            vrow = idx_v[b, pl.ds(i * 16, 16)]
            gidx = vrow * _D
            sidx = iota64 + i * (16 * _D)
            for j in range(_D):
                col = plsc.load_gather(table_v, [gidx + j])
                plsc.store_scatter(rows_v.at[b], [sidx + j], col)
            return carry

        lax.fori_loop(0, _BLK, blk, 0)

    fire_idx(0, 0)

    def pair(p, carry):
        g0 = p * 2
        for b in range(2):
            gg = g0 + b
            nb = 1 - b
            wait_idx(b)

            @pl.when(gg >= 2)
            def _():
                wait_out(b)

            @pl.when(gg + 1 < _G)
            def _():
                fire_idx(gg + 1, nb)

            compute_group(b)
            fire_out(gg, b)
        return carry

    lax.fori_loop(0, _G // 2, pair, 0)
    wait_out(0)
    wait_out(1)


def kernel(input, weight):
    idx = input.reshape(_B_TOTAL).astype(jnp.int32)
    table = weight.reshape(_V * _D)
    mesh = plsc.VectorSubcoreMesh(core_axis_name="c", subcore_axis_name="s")
    call = pl.kernel(
        _emb_body,
        out_type=jax.ShapeDtypeStruct((_B_TOTAL * _D,), jnp.float32),
        mesh=mesh,
        scratch_types=[
            pltpu.VMEM((_V * _D,), jnp.float32),
            pltpu.VMEM((2, _GRP), jnp.int32),
            pltpu.VMEM((2, _GRP * _D), jnp.float32),
            pltpu.SemaphoreType.DMA,
            pltpu.SemaphoreType.DMA,
        ],
        compiler_params=pltpu.CompilerParams(
            use_tc_tiling_on_sc=False, needs_layout_passes=False
        ),
    )
    out = call(idx, table)
    return out.reshape(16384, 200, _D)


# speedup vs baseline: 1.0383x; 1.0383x over previous
import jax
import jax.numpy as jnp
from jax import lax
from jax.experimental import pallas as pl
from jax.experimental.pallas import tpu as pltpu
from jax.experimental.pallas import tpu_sc as plsc

_B_TOTAL = 16384 * 200
_D = 64
_V = 1000
_NC, _NS = 2, 16
_NW = _NC * _NS
_B_PER_W = _B_TOTAL // _NW
_GRP = 256
_G = _B_PER_W // _GRP
_BLK = _GRP // 16


def _emb_body(idx_hbm, table_hbm, out_hbm, table_v, idx_v, rows_v, isem, osem):
    sid = lax.axis_index("s")
    wid = sid * _NC + lax.axis_index("c")
    base = wid * _B_PER_W

    pltpu.sync_copy(table_hbm, table_v)

    iota64 = lax.iota(jnp.int32, 16) * 64

    def fire_idx(g, b):
        pltpu.async_copy(idx_hbm.at[pl.ds(base + g * _GRP, _GRP)], idx_v.at[b], isem)

    def wait_idx(b):
        pltpu.make_async_copy(idx_hbm.at[pl.ds(0, _GRP)], idx_v.at[b], isem).wait()

    def fire_out(g, b):
        pltpu.async_copy(
            rows_v.at[b], out_hbm.at[pl.ds((base + g * _GRP) * _D, _GRP * _D)], osem
        )

    def wait_out(b):
        pltpu.make_async_copy(
            rows_v.at[b], out_hbm.at[pl.ds(0, _GRP * _D)], osem
        ).wait()

    def compute_group(b):
        @plsc.parallel_loop(0, _BLK, unroll=2)
        def blk(i):
            vrow = idx_v[b, pl.ds(i * 16, 16)]
            gidx = vrow * _D
            sidx = iota64 + i * (16 * _D)
            for j0 in range(0, _D, 32):
                cols = [plsc.load_gather(table_v, [gidx + (j0 + t)]) for t in range(32)]
                for t in range(32):
                    plsc.store_scatter(rows_v.at[b], [sidx + (j0 + t)], cols[t])

    fire_idx(0, 0)

    def pair(p, carry):
        g0 = p * 2
        for b in range(2):
            gg = g0 + b
            nb = 1 - b
            wait_idx(b)

            @pl.when(gg >= 2)
            def _():
                wait_out(b)

            @pl.when(gg + 1 < _G)
            def _():
                fire_idx(gg + 1, nb)

            compute_group(b)
            fire_out(gg, b)
        return carry

    lax.fori_loop(0, _G // 2, pair, 0)
    wait_out(0)
    wait_out(1)


def kernel(input, weight):
    idx = input.reshape(_B_TOTAL).astype(jnp.int32)
    table = weight.reshape(_V * _D)
    mesh = plsc.VectorSubcoreMesh(core_axis_name="c", subcore_axis_name="s")
    call = pl.kernel(
        _emb_body,
        out_type=jax.ShapeDtypeStruct((_B_TOTAL * _D,), jnp.float32),
        mesh=mesh,
        scratch_types=[
            pltpu.VMEM((_V * _D,), jnp.float32),
            pltpu.VMEM((2, _GRP), jnp.int32),
            pltpu.VMEM((2, _GRP * _D), jnp.float32),
            pltpu.SemaphoreType.DMA,
            pltpu.SemaphoreType.DMA,
        ],
        compiler_params=pltpu.CompilerParams(
            use_tc_tiling_on_sc=False, needs_layout_passes=False
        ),
    )
    out = call(idx, table)
    return out.reshape(16384, 200, _D)


# speedup vs baseline: 3.9348x; 3.7897x over previous
import jax
import jax.numpy as jnp
from jax import lax
from jax.experimental import pallas as pl
from jax.experimental.pallas import tpu as pltpu
from jax.experimental.pallas import tpu_sc as plsc

_B_TOTAL = 16384 * 200
_D = 64
_V = 1000
_NC, _NS = 2, 16
_NW = _NC * _NS
_B_PER_W = _B_TOTAL // _NW
_K = 4
_G = _B_PER_W // (_K * 128)
_ROWS_PER_W = _B_PER_W // 128


def _emb_body(idx_hbm, table_hbm, out_hbm, table_sh, idx_v, rows_v, gsem, osem):
    sid = lax.axis_index("s")
    wid = sid * _NC + lax.axis_index("c")
    row_base = wid * _ROWS_PER_W

    pltpu.sync_copy(table_hbm, table_sh)

    def fire_group(g, b):
        r0 = row_base + g * _K
        pltpu.sync_copy(idx_hbm.at[pl.ds(r0, _K)], idx_v.at[b])
        for j in range(_K):
            pltpu.async_copy(table_sh.at[idx_v.at[b].at[j]], rows_v.at[b].at[j], gsem)

    def drain_group(b):
        for j in range(_K):
            pltpu.make_async_copy(
                table_sh.at[idx_v.at[b].at[j]], rows_v.at[b].at[j], gsem
            ).wait()

    def drain_out(b):
        pltpu.make_async_copy(
            rows_v.at[b], out_hbm.at[pl.ds(0, _K)], osem
        ).wait()

    fire_group(0, 0)

    def pair(p, carry):
        g0 = p * 2
        for b in range(2):
            gg = g0 + b
            nb = 1 - b
            drain_group(b)
            pltpu.async_copy(
                rows_v.at[b], out_hbm.at[pl.ds(row_base + gg * _K, _K)], osem
            )

            @pl.when(gg >= 1)
            def _():
                drain_out(nb)

            @pl.when(gg + 1 < _G)
            def _():
                fire_group(gg + 1, nb)
        return carry

    lax.fori_loop(0, _G // 2, pair, 0)
    drain_out(1)


def kernel(input, weight):
    idx = input.reshape(_B_TOTAL // 128, 128).astype(jnp.int32)
    mesh = plsc.VectorSubcoreMesh(core_axis_name="c", subcore_axis_name="s")
    call = pl.kernel(
        _emb_body,
        out_type=jax.ShapeDtypeStruct((_B_TOTAL // 128, 128, _D), jnp.float32),
        mesh=mesh,
        scratch_types=[
            pltpu.VMEM_SHARED((_V, _D), jnp.float32),
            pltpu.VMEM((2, _K, 128), jnp.int32),
            pltpu.VMEM((2, _K, 128, _D), jnp.float32),
            pltpu.SemaphoreType.DMA,
            pltpu.SemaphoreType.DMA,
        ],
        compiler_params=pltpu.CompilerParams(use_tc_tiling_on_sc=False),
    )
    out = call(idx, weight)
    return out.reshape(16384, 200, _D)


# speedup vs baseline: 4.0035x; 1.0174x over previous
import jax
import jax.numpy as jnp
from jax import lax
from jax.experimental import pallas as pl
from jax.experimental.pallas import tpu as pltpu
from jax.experimental.pallas import tpu_sc as plsc

_N = 16384
_M = 200
_D = 64
_V = 1000
_NC, _NS = 2, 16
_NW = _NC * _NS
_ROWS_PER_W = _N // _NW
_K = 4
_G = _ROWS_PER_W // _K
_CH = ((0, 128), (128, _M - 128))


def _emb_body(idx_hbm, table_hbm, out_hbm, table_sh, idx_v, rows_v, gsem, osem):
    sid = lax.axis_index("s")
    wid = sid * _NC + lax.axis_index("c")
    row_base = wid * _ROWS_PER_W

    pltpu.sync_copy(table_hbm, table_sh)

    def fire_group(g, b):
        r0 = row_base + g * _K
        pltpu.sync_copy(idx_hbm.at[pl.ds(r0, _K)], idx_v.at[b])
        for j in range(_K):
            for (o, w) in _CH:
                pltpu.async_copy(
                    table_sh.at[idx_v.at[b].at[j].at[pl.ds(o, w)]],
                    rows_v.at[b].at[j].at[pl.ds(o, w)],
                    gsem,
                )

    def drain_group(b):
        for j in range(_K):
            for (o, w) in _CH:
                pltpu.make_async_copy(
                    table_sh.at[idx_v.at[b].at[j].at[pl.ds(o, w)]],
                    rows_v.at[b].at[j].at[pl.ds(o, w)],
                    gsem,
                ).wait()

    def drain_out(b):
        pltpu.make_async_copy(
            rows_v.at[b], out_hbm.at[pl.ds(0, _K)], osem
        ).wait()

    fire_group(0, 0)

    def pair(p, carry):
        g0 = p * 2
        for b in range(2):
            gg = g0 + b
            nb = 1 - b
            drain_group(b)
            pltpu.async_copy(
                rows_v.at[b], out_hbm.at[pl.ds(row_base + gg * _K, _K)], osem
            )

            @pl.when(gg >= 1)
            def _():
                drain_out(nb)

            @pl.when(gg + 1 < _G)
            def _():
                fire_group(gg + 1, nb)
        return carry

    lax.fori_loop(0, _G // 2, pair, 0)
    drain_out(1)


def kernel(input, weight):
    mesh = plsc.VectorSubcoreMesh(core_axis_name="c", subcore_axis_name="s")
    call = pl.kernel(
        _emb_body,
        out_type=jax.ShapeDtypeStruct((_N, _M, _D), jnp.float32),
        mesh=mesh,
        scratch_types=[
            pltpu.VMEM_SHARED((_V, _D), jnp.float32),
            pltpu.VMEM((2, _K, _M), jnp.int32),
            pltpu.VMEM((2, _K, _M, _D), jnp.float32),
            pltpu.SemaphoreType.DMA,
            pltpu.SemaphoreType.DMA,
        ],
        compiler_params=pltpu.CompilerParams(use_tc_tiling_on_sc=False),
    )
    return call(input.astype(jnp.int32), weight)
